# baseline (speedup 1.0000x reference)
import functools

import jax
import jax.numpy as jnp
from jax import lax
from jax.experimental import pallas as pl
from jax.experimental.pallas import tpu as pltpu
from jax.experimental.pallas import tpu_sc as plsc

NUM_NODES = 10000
N_PAD = 10240
CH = 128
E_TOTAL = 320000
SCHUNK = 512
NCH = 640
E_PAD = NCH * SCHUNK
NCH_HALF = NCH // 2
PKW = 3 * SCHUNK
STAGE = 1152
FLUSH = 128
OWN = N_PAD // 16
DEG_PAD = N_PAD + 2048
DEG_PER_TILE = DEG_PAD // 16

_mesh = plsc.VectorSubcoreMesh(core_axis_name="c", subcore_axis_name="s")


@functools.partial(
    pl.kernel,
    out_type=jax.ShapeDtypeStruct((2, N_PAD), jnp.float32),
    mesh=_mesh,
    compiler_params=pltpu.CompilerParams(needs_layout_passes=False),
    scratch_types=[
        pltpu.VMEM((PKW,), jnp.int32),
        pltpu.VMEM((PKW,), jnp.int32),
        pltpu.VMEM((FLUSH,), jnp.int32),
        pltpu.VMEM((FLUSH,), jnp.float32),
        pltpu.VMEM_SHARED((DEG_PAD,), jnp.float32),
        pltpu.SemaphoreType.DMA,
        pltpu.SemaphoreType.DMA,
    ],
)
def _deg_kernel(pk_hbm, zeros_hbm, deg_out, pb0, pb1, kd_v, kw_v, deg_sh,
                sem0, sem1):
    c = lax.axis_index("c")
    s = lax.axis_index("s")
    t = c * 16 + s
    pltpu.sync_copy(zeros_hbm, deg_sh.at[pl.ds(s * DEG_PER_TILE, DEG_PER_TILE)])
    plsc.subcore_barrier()
    rows_per_tile = NCH // 32
    base = t * rows_per_tile

    def process(buf):
        for q in range(4):
            for g in range(8):
                sl = pl.ds(g * 16, 16)
                kd_v[sl] = buf[pl.ds(SCHUNK + q * 128 + g * 16, 16)]
                kw_v[sl] = plsc.bitcast(
                    buf[pl.ds(2 * SCHUNK + q * 128 + g * 16, 16)], jnp.float32)
            pltpu.sync_copy(kw_v, deg_sh.at[kd_v], add=True)

    pltpu.async_copy(pk_hbm.at[base], pb0, sem0)

    def body(i, carry):
        pltpu.async_copy(pk_hbm.at[base + 2 * i + 1], pb1, sem1)
        pltpu.make_async_copy(pk_hbm.at[0], pb0, sem0).wait()
        process(pb0)
        pltpu.async_copy(pk_hbm.at[base + 2 * i + 2], pb0, sem0)
        pltpu.make_async_copy(pk_hbm.at[0], pb1, sem1).wait()
        process(pb1)
        return carry

    lax.fori_loop(0, rows_per_tile // 2, body, 0)
    pltpu.make_async_copy(pk_hbm.at[0], pb0, sem0).wait()
    plsc.subcore_barrier()
    pltpu.sync_copy(deg_sh.at[pl.ds(s * OWN, OWN)],
                    deg_out.at[c, pl.ds(s * OWN, OWN)])


def _mm_body(x_ref, w_ref, degp_ref, y_ref, dinv_ref):
    deg = degp_ref[0] + degp_ref[1]
    dinv_ref[...] = jnp.where(
        deg > 0.0, lax.rsqrt(jnp.maximum(deg, 1e-30)), 0.0)
    y_ref[...] = jnp.dot(x_ref[...], w_ref[...],
                         preferred_element_type=jnp.float32)


def _matmul_dinv(x_pad, W, degp):
    grid = N_PAD // 1024
    return pl.pallas_call(
        _mm_body,
        grid=(grid,),
        in_specs=[
            pl.BlockSpec((1024, CH), lambda i: (i, 0)),
            pl.BlockSpec((CH, CH), lambda i: (0, 0)),
            pl.BlockSpec((2, 8, CH), lambda i: (0, i, 0)),
        ],
        out_specs=[
            pl.BlockSpec((1024, CH), lambda i: (i, 0)),
            pl.BlockSpec((8, CH), lambda i: (i, 0)),
        ],
        out_shape=[
            jax.ShapeDtypeStruct((N_PAD, CH), jnp.float32),
            jax.ShapeDtypeStruct((N_PAD // CH, CH), jnp.float32),
        ],
    )(x_pad, W, degp)


def _issue_batch(cnt, masked, lo, dinv_v, gidx_v, bdst_v, st_src, st_dst,
                 st_w, norm_v, rows_v, y_hbm, sem):
    for g in range(FLUSH // 16):
        sl = pl.ds(g * 16, 16)
        sv = st_src[sl]
        dv = st_dst[sl]
        gidx_v[sl] = sv
        bdst_v[sl] = dv
        nv = (plsc.load_gather(dinv_v, [sv]) * st_w[sl] *
              plsc.load_gather(dinv_v, [dv]))
        if masked:
            lane = jnp.arange(g * 16, g * 16 + 16, dtype=jnp.int32)
            nv = jnp.where(lane < cnt, nv, 0.0)
        norm_v[sl] = nv
    pltpu.async_copy(y_hbm.at[gidx_v], rows_v, sem)


def _complete_batch(lo, gidx_v, bdst_v, norm_v, rows_v, acc_v, y_hbm, sem):
    _COL = [jnp.arange(j * 16, j * 16 + 16, dtype=jnp.int32) for j in range(8)]
    pltpu.make_async_copy(y_hbm.at[gidx_v], rows_v, sem).wait()

    def acc_step(st, carry):
        for k in range(16):
            ev = jnp.full((16,), st * 16 + k, dtype=jnp.int32)
            nsplat = plsc.load_gather(norm_v, [ev])
            dsplat = plsc.load_gather(bdst_v, [ev]) - lo
            for j in range(8):
                val = plsc.load_gather(rows_v, [ev, _COL[j]]) * nsplat
                plsc.addupdate_scatter(acc_v, [dsplat, _COL[j]], val)
        return carry

    lax.fori_loop(0, FLUSH // 16, acc_step, 0)


@functools.partial(
    pl.kernel,
    out_type=jax.ShapeDtypeStruct((2, N_PAD, CH), jnp.float32),
    mesh=_mesh,
    compiler_params=pltpu.CompilerParams(needs_layout_passes=False),
    scratch_types=[
        pltpu.VMEM((N_PAD,), jnp.float32),
        pltpu.VMEM((PKW,), jnp.int32),
        pltpu.VMEM((PKW,), jnp.int32),
        pltpu.VMEM((STAGE,), jnp.int32),
        pltpu.VMEM((STAGE,), jnp.int32),
        pltpu.VMEM((STAGE,), jnp.float32),
        pltpu.VMEM((FLUSH,), jnp.int32),
        pltpu.VMEM((FLUSH,), jnp.int32),
        pltpu.VMEM((FLUSH,), jnp.float32),
        pltpu.VMEM((FLUSH, CH), jnp.float32),
        pltpu.VMEM((OWN, CH), jnp.float32),
        pltpu.SemaphoreType.DMA,
        pltpu.SemaphoreType.DMA,
        pltpu.SemaphoreType.DMA,
    ],
)
def _msg_kernel(y_hbm, dinv_hbm, pk_hbm, zrow_hbm, acc_out,
                dinv_v, pb0, pb1, st_src, st_dst, st_w,
                gidx_v, bdst_v, norm_v, rows_v, acc_v, sem0, sem1, semg):
    c = lax.axis_index("c")
    s = lax.axis_index("s")
    lo = s * OWN
    hi = lo + OWN
    pltpu.sync_copy(zrow_hbm, acc_v)
    pltpu.sync_copy(dinv_hbm, dinv_v)
    zi = jnp.zeros((16,), jnp.int32)
    zf = jnp.zeros((16,), jnp.float32)
    lov = jnp.full((16,), lo, dtype=jnp.int32)
    for g in range(STAGE // 16):
        sl = pl.ds(g * 16, 16)
        st_src[sl] = zi
        st_dst[sl] = lov
        st_w[sl] = zf
    base = c * NCH_HALF

    def scan(buf, cnt):
        for g in range(SCHUNK // 16):
            dv = buf[pl.ds(SCHUNK + g * 16, 16)]
            m = (dv >= lo) & (dv < hi)
            plsc.store_compressed(st_src.at[pl.ds(cnt, 16)],
                                  buf[pl.ds(g * 16, 16)], mask=m)
            plsc.store_compressed(st_dst.at[pl.ds(cnt, 16)], dv, mask=m)
            wv = plsc.bitcast(buf[pl.ds(2 * SCHUNK + g * 16, 16)], jnp.float32)
            plsc.store_compressed(st_w.at[pl.ds(cnt, 16)], wv, mask=m)
            cnt = cnt + jnp.max(plsc.all_reduce_population_count(m))
        return cnt

    pltpu.async_copy(pk_hbm.at[base], pb0, sem0)
    _issue_batch(0, False, lo, dinv_v, gidx_v, bdst_v, st_src, st_dst,
                 st_w, norm_v, rows_v, y_hbm, semg)

    def scan_chunk(i, cnt):
        pltpu.async_copy(pk_hbm.at[base + 2 * i + 1], pb1, sem1)
        pltpu.make_async_copy(pk_hbm.at[0], pb0, sem0).wait()
        cnt = scan(pb0, cnt)
        pltpu.async_copy(pk_hbm.at[base + 2 * i + 2], pb0, sem0)
        pltpu.make_async_copy(pk_hbm.at[0], pb1, sem1).wait()
        cnt = scan(pb1, cnt)

        def do_flush(n):
            _complete_batch(lo, gidx_v, bdst_v, norm_v, rows_v, acc_v,
                            y_hbm, semg)
            _issue_batch(n, False, lo, dinv_v, gidx_v, bdst_v, st_src,
                         st_dst, st_w, norm_v, rows_v, y_hbm, semg)
            for g in range((STAGE - FLUSH) // 16):
                sf = pl.ds(FLUSH + g * 16, 16)
                stl = pl.ds(g * 16, 16)
                st_src[stl] = st_src[sf]
                st_dst[stl] = st_dst[sf]
                st_w[stl] = st_w[sf]
            return n - FLUSH

        return lax.while_loop(lambda n: n >= FLUSH, do_flush, cnt)

    cnt = lax.fori_loop(0, NCH_HALF // 2, scan_chunk, jnp.int32(0))
    pltpu.make_async_copy(pk_hbm.at[0], pb0, sem0).wait()
    _complete_batch(lo, gidx_v, bdst_v, norm_v, rows_v, acc_v, y_hbm, semg)
    _issue_batch(cnt, True, lo, dinv_v, gidx_v, bdst_v, st_src, st_dst,
                 st_w, norm_v, rows_v, y_hbm, semg)
    _complete_batch(lo, gidx_v, bdst_v, norm_v, rows_v, acc_v, y_hbm, semg)
    pltpu.sync_copy(acc_v, acc_out.at[c, pl.ds(lo, OWN)])


def _relu_body(p_ref, o_ref):
    o_ref[...] = jnp.maximum(p_ref[0] + p_ref[1], 0.0)


def _relu_add(parts):
    grid = N_PAD // 512
    return pl.pallas_call(
        _relu_body,
        grid=(grid,),
        in_specs=[pl.BlockSpec((2, 512, CH), lambda i: (0, i, 0))],
        out_specs=pl.BlockSpec((512, CH), lambda i: (i, 0)),
        out_shape=jax.ShapeDtypeStruct((N_PAD, CH), jnp.float32),
    )(parts)


def kernel(x, u_edge_index, u_edge_weight, v_edge_index, v_edge_weight, W):
    src = jnp.concatenate([u_edge_index[0], v_edge_index[0]]).astype(jnp.int32)
    dst = jnp.concatenate([u_edge_index[1], v_edge_index[1]]).astype(jnp.int32)
    w = jnp.concatenate([u_edge_weight, v_edge_weight]).astype(jnp.float32)
    pad = E_PAD - E_TOTAL
    src = jnp.pad(src, (0, pad))
    dst = jnp.pad(dst, (0, pad), constant_values=N_PAD)
    w = jnp.pad(w, (0, pad))
    packed = jnp.concatenate(
        [src.reshape(NCH, SCHUNK), dst.reshape(NCH, SCHUNK),
         lax.bitcast_convert_type(w, jnp.int32).reshape(NCH, SCHUNK)], axis=1)
    packed = jnp.pad(packed, ((0, 1), (0, 0)))

    x_pad = jnp.pad(x, ((0, N_PAD - NUM_NODES), (0, 0)))
    zeros1 = jnp.zeros((DEG_PER_TILE,), jnp.float32)
    zeros2 = jnp.zeros((OWN, CH), jnp.float32)

    degp = _deg_kernel(packed, zeros1)
    degp3 = degp.reshape(2, N_PAD // CH, CH)
    y, dinv2 = _matmul_dinv(x_pad, W, degp3)
    dinv = dinv2.reshape(N_PAD)
    parts = _msg_kernel(y, dinv, packed, zeros2)
    out = _relu_add(parts)
    return out[:NUM_NODES]


# speedup vs baseline: 1.1042x; 1.1042x over previous
import functools

import jax
import jax.numpy as jnp
from jax import lax
from jax.experimental import pallas as pl
from jax.experimental.pallas import tpu as pltpu
from jax.experimental.pallas import tpu_sc as plsc

NUM_NODES = 10000
N_PAD = 10240
CH = 128
E_TOTAL = 320000
SCHUNK = 512
NCH = 640
E_PAD = NCH * SCHUNK
NCH_HALF = NCH // 2
PKW = 3 * SCHUNK
STAGE = 1152
FLUSH = 128
OWN = N_PAD // 16
DEG_PAD = N_PAD + 2048
DEG_PER_TILE = DEG_PAD // 16

_mesh = plsc.VectorSubcoreMesh(core_axis_name="c", subcore_axis_name="s")


@functools.partial(
    pl.kernel,
    out_type=jax.ShapeDtypeStruct((2, N_PAD), jnp.float32),
    mesh=_mesh,
    compiler_params=pltpu.CompilerParams(needs_layout_passes=False),
    scratch_types=[
        pltpu.VMEM((PKW,), jnp.int32),
        pltpu.VMEM((PKW,), jnp.int32),
        pltpu.VMEM((FLUSH,), jnp.int32),
        pltpu.VMEM((FLUSH,), jnp.float32),
        pltpu.VMEM_SHARED((DEG_PAD,), jnp.float32),
        pltpu.SemaphoreType.DMA,
        pltpu.SemaphoreType.DMA,
    ],
)
def _deg_kernel(pk_hbm, zeros_hbm, deg_out, pb0, pb1, kd_v, kw_v, deg_sh,
                sem0, sem1):
    c = lax.axis_index("c")
    s = lax.axis_index("s")
    t = c * 16 + s
    pltpu.sync_copy(zeros_hbm, deg_sh.at[pl.ds(s * DEG_PER_TILE, DEG_PER_TILE)])
    plsc.subcore_barrier()
    rows_per_tile = NCH // 32
    base = t * rows_per_tile

    def process(buf):
        for q in range(4):
            for g in range(8):
                sl = pl.ds(g * 16, 16)
                kd_v[sl] = buf[pl.ds(SCHUNK + q * 128 + g * 16, 16)]
                kw_v[sl] = plsc.bitcast(
                    buf[pl.ds(2 * SCHUNK + q * 128 + g * 16, 16)], jnp.float32)
            pltpu.sync_copy(kw_v, deg_sh.at[kd_v], add=True)

    pltpu.async_copy(pk_hbm.at[base], pb0, sem0)

    def body(i, carry):
        pltpu.async_copy(pk_hbm.at[base + 2 * i + 1], pb1, sem1)
        pltpu.make_async_copy(pk_hbm.at[0], pb0, sem0).wait()
        process(pb0)
        pltpu.async_copy(pk_hbm.at[base + 2 * i + 2], pb0, sem0)
        pltpu.make_async_copy(pk_hbm.at[0], pb1, sem1).wait()
        process(pb1)
        return carry

    lax.fori_loop(0, rows_per_tile // 2, body, 0)
    pltpu.make_async_copy(pk_hbm.at[0], pb0, sem0).wait()
    plsc.subcore_barrier()
    pltpu.sync_copy(deg_sh.at[pl.ds(s * OWN, OWN)],
                    deg_out.at[c, pl.ds(s * OWN, OWN)])


def _mm_body(x_ref, w_ref, degp_ref, y_ref, dinv_ref):
    deg = degp_ref[0] + degp_ref[1]
    dinv_ref[...] = jnp.where(
        deg > 0.0, lax.rsqrt(jnp.maximum(deg, 1e-30)), 0.0)
    y_ref[...] = jnp.dot(x_ref[...], w_ref[...],
                         preferred_element_type=jnp.float32)


def _matmul_dinv(x_pad, W, degp):
    grid = N_PAD // 1024
    return pl.pallas_call(
        _mm_body,
        grid=(grid,),
        in_specs=[
            pl.BlockSpec((1024, CH), lambda i: (i, 0)),
            pl.BlockSpec((CH, CH), lambda i: (0, 0)),
            pl.BlockSpec((2, 8, CH), lambda i: (0, i, 0)),
        ],
        out_specs=[
            pl.BlockSpec((1024, CH), lambda i: (i, 0)),
            pl.BlockSpec((8, CH), lambda i: (i, 0)),
        ],
        out_shape=[
            jax.ShapeDtypeStruct((N_PAD, CH), jnp.float32),
            jax.ShapeDtypeStruct((N_PAD // CH, CH), jnp.float32),
        ],
    )(x_pad, W, degp)


def _flush_batch(cnt, masked, lo, dinv_v, gidx_v, st_src, st_dst, st_w,
                 norm_v, rows_v, acc_v, y_hbm, sem):
    _COL = [jnp.arange(j * 16, j * 16 + 16, dtype=jnp.int32) for j in range(8)]
    for g in range(FLUSH // 16):
        sl = pl.ds(g * 16, 16)
        gidx_v[sl] = st_src[sl]
    pltpu.async_copy(y_hbm.at[gidx_v], rows_v, sem).wait()
    for g in range(FLUSH // 16):
        sl = pl.ds(g * 16, 16)
        nv = (plsc.load_gather(dinv_v, [gidx_v[sl]]) * st_w[sl] *
              plsc.load_gather(dinv_v, [st_dst[sl]]))
        if masked:
            lane = jnp.arange(g * 16, g * 16 + 16, dtype=jnp.int32)
            nv = jnp.where(lane < cnt, nv, 0.0)
        norm_v[sl] = nv

    def acc_step(st, carry):
        for k in range(16):
            ev = jnp.full((16,), st * 16 + k, dtype=jnp.int32)
            nsplat = plsc.load_gather(norm_v, [ev])
            dsplat = plsc.load_gather(st_dst, [ev]) - lo
            for j in range(8):
                val = plsc.load_gather(rows_v, [ev, _COL[j]]) * nsplat
                plsc.addupdate_scatter(acc_v, [dsplat, _COL[j]], val)
        return carry

    lax.fori_loop(0, FLUSH // 16, acc_step, 0)


@functools.partial(
    pl.kernel,
    out_type=jax.ShapeDtypeStruct((2, N_PAD, CH), jnp.float32),
    mesh=_mesh,
    compiler_params=pltpu.CompilerParams(needs_layout_passes=False),
    scratch_types=[
        pltpu.VMEM((N_PAD,), jnp.float32),
        pltpu.VMEM((PKW,), jnp.int32),
        pltpu.VMEM((PKW,), jnp.int32),
        pltpu.VMEM((STAGE,), jnp.int32),
        pltpu.VMEM((STAGE,), jnp.int32),
        pltpu.VMEM((STAGE,), jnp.float32),
        pltpu.VMEM((FLUSH,), jnp.int32),
        pltpu.VMEM((FLUSH,), jnp.float32),
        pltpu.VMEM((FLUSH, CH), jnp.float32),
        pltpu.VMEM((OWN, CH), jnp.float32),
        pltpu.SemaphoreType.DMA,
        pltpu.SemaphoreType.DMA,
        pltpu.SemaphoreType.DMA,
    ],
)
def _msg_kernel(y_hbm, dinv_hbm, pk_hbm, zrow_hbm, acc_out,
                dinv_v, pb0, pb1, st_src, st_dst, st_w,
                gidx_v, norm_v, rows_v, acc_v, sem0, sem1, semg):
    c = lax.axis_index("c")
    s = lax.axis_index("s")
    lo = s * OWN
    hi = lo + OWN
    pltpu.sync_copy(zrow_hbm, acc_v)
    pltpu.sync_copy(dinv_hbm, dinv_v)
    zi = jnp.zeros((16,), jnp.int32)
    zf = jnp.zeros((16,), jnp.float32)
    lov = jnp.full((16,), lo, dtype=jnp.int32)
    for g in range(STAGE // 16):
        sl = pl.ds(g * 16, 16)
        st_src[sl] = zi
        st_dst[sl] = lov
        st_w[sl] = zf
    base = c * NCH_HALF

    def scan(buf, cnt):
        for g in range(SCHUNK // 16):
            dv = buf[pl.ds(SCHUNK + g * 16, 16)]
            m = (dv >= lo) & (dv < hi)
            plsc.store_compressed(st_src.at[pl.ds(cnt, 16)],
                                  buf[pl.ds(g * 16, 16)], mask=m)
            plsc.store_compressed(st_dst.at[pl.ds(cnt, 16)], dv, mask=m)
            wv = plsc.bitcast(buf[pl.ds(2 * SCHUNK + g * 16, 16)], jnp.float32)
            plsc.store_compressed(st_w.at[pl.ds(cnt, 16)], wv, mask=m)
            cnt = cnt + plsc.all_reduce_population_count(m)[0]
        return cnt

    pltpu.async_copy(pk_hbm.at[base], pb0, sem0)

    def scan_chunk(i, cnt):
        pltpu.async_copy(pk_hbm.at[base + 2 * i + 1], pb1, sem1)
        pltpu.make_async_copy(pk_hbm.at[0], pb0, sem0).wait()
        cnt = scan(pb0, cnt)
        pltpu.async_copy(pk_hbm.at[base + 2 * i + 2], pb0, sem0)
        pltpu.make_async_copy(pk_hbm.at[0], pb1, sem1).wait()
        cnt = scan(pb1, cnt)

        def do_flush(n):
            _flush_batch(n, False, lo, dinv_v, gidx_v, st_src, st_dst, st_w,
                         norm_v, rows_v, acc_v, y_hbm, semg)
            for g in range((STAGE - FLUSH) // 16):
                sf = pl.ds(FLUSH + g * 16, 16)
                stl = pl.ds(g * 16, 16)
                st_src[stl] = st_src[sf]
                st_dst[stl] = st_dst[sf]
                st_w[stl] = st_w[sf]
            return n - FLUSH

        return lax.while_loop(lambda n: n >= FLUSH, do_flush, cnt)

    cnt = lax.fori_loop(0, NCH_HALF // 2, scan_chunk, jnp.int32(0))
    pltpu.make_async_copy(pk_hbm.at[0], pb0, sem0).wait()
    _flush_batch(cnt, True, lo, dinv_v, gidx_v, st_src, st_dst, st_w,
                 norm_v, rows_v, acc_v, y_hbm, semg)
    pltpu.sync_copy(acc_v, acc_out.at[c, pl.ds(lo, OWN)])


def _relu_body(p_ref, o_ref):
    o_ref[...] = jnp.maximum(p_ref[0] + p_ref[1], 0.0)


def _relu_add(parts):
    grid = N_PAD // 512
    return pl.pallas_call(
        _relu_body,
        grid=(grid,),
        in_specs=[pl.BlockSpec((2, 512, CH), lambda i: (0, i, 0))],
        out_specs=pl.BlockSpec((512, CH), lambda i: (i, 0)),
        out_shape=jax.ShapeDtypeStruct((N_PAD, CH), jnp.float32),
    )(parts)


def kernel(x, u_edge_index, u_edge_weight, v_edge_index, v_edge_weight, W):
    src = jnp.concatenate([u_edge_index[0], v_edge_index[0]]).astype(jnp.int32)
    dst = jnp.concatenate([u_edge_index[1], v_edge_index[1]]).astype(jnp.int32)
    w = jnp.concatenate([u_edge_weight, v_edge_weight]).astype(jnp.float32)
    pad = E_PAD - E_TOTAL
    src = jnp.pad(src, (0, pad))
    dst = jnp.pad(dst, (0, pad), constant_values=N_PAD)
    w = jnp.pad(w, (0, pad))
    packed = jnp.concatenate(
        [src.reshape(NCH, SCHUNK), dst.reshape(NCH, SCHUNK),
         lax.bitcast_convert_type(w, jnp.int32).reshape(NCH, SCHUNK)], axis=1)
    packed = jnp.pad(packed, ((0, 1), (0, 0)))

    x_pad = jnp.pad(x, ((0, N_PAD - NUM_NODES), (0, 0)))
    zeros1 = jnp.zeros((DEG_PER_TILE,), jnp.float32)
    zeros2 = jnp.zeros((OWN, CH), jnp.float32)

    degp = _deg_kernel(packed, zeros1)
    degp3 = degp.reshape(2, N_PAD // CH, CH)
    y, dinv2 = _matmul_dinv(x_pad, W, degp3)
    dinv = dinv2.reshape(N_PAD)
    parts = _msg_kernel(y, dinv, packed, zeros2)
    out = _relu_add(parts)
    return out[:NUM_NODES]
